# baseline (speedup 1.0000x reference)
import jax
import jax.numpy as jnp
from jax.experimental import pallas as pl
from jax.experimental.pallas import tpu as pltpu

SAMPLES_PER_BLOCK = 8


def kernel(x, A, W1, b1, W2, b2):
    n, t, v, ci = x.shape
    k = A.shape[0]
    h = W1.shape[0] // k
    co = W2.shape[0] // k
    p, r = v * ci, v * co
    q = v * h
    tp = (t + 7) // 8 * 8 + 4
    pp = 128
    rp = (r + 127) // 128 * 128
    bn = SAMPLES_PER_BLOCK

    W1r = W1.reshape(k, h, ci).transpose(0, 2, 1)
    W2r = W2.reshape(k, co, h).transpose(0, 2, 1)
    M1 = (A[:, :, None, :, None] * W1r[:, None, :, None, :]).sum(0)
    M1 = M1.reshape(p, q)
    M2 = (A[:, :, None, :, None] * W2r[:, None, :, None, :]).sum(0)
    M2 = M2.reshape(q, r)
    S = A.sum(axis=1)
    b1r = b1.reshape(k, h)
    b2r = b2.reshape(k, co)
    B1 = (S.T[:, :, None] * b1r[None, :, :]).sum(1).reshape(1, q)
    b2eff = (S.T[:, :, None] * b2r[None, :, :]).sum(1).reshape(1, r)

    m1a = jnp.concatenate([M1, B1], axis=0)

    X3 = x.reshape(n, t, p)
    Xp = jax.lax.pad(
        X3, jnp.float32(0), ((0, 0, 0), (0, tp - t, 0), (0, pp - p, 0)))

    def gemm_fold_kernel(x_ref, m1a_ref, m2_ref, b2_ref, out_ref,
                         ms_ref, bs_ref):
        i = pl.program_id(0)

        @pl.when(i == 0)
        def _():
            mf = jnp.dot(m1a_ref[...], m2_ref[...],
                         preferred_element_type=jnp.float32)
            ms_ref[...] = jnp.zeros((pp, rp), jnp.bfloat16)
            ms_ref[0:p, 0:r] = mf[0:p].astype(jnp.bfloat16)
            bs_ref[...] = jnp.zeros((1, rp), jnp.float32)
            bs_ref[0:1, 0:r] = mf[p:p + 1] + b2_ref[...]

        mb = ms_ref[...]
        bb = bs_ref[...]
        for j in range(bn):
            xb = x_ref[j].astype(jnp.bfloat16)
            out_ref[j] = (
                jnp.dot(xb, mb, preferred_element_type=jnp.float32) + bb)

    padded = pl.pallas_call(
        gemm_fold_kernel,
        grid=(n // bn,),
        in_specs=[
            pl.BlockSpec((bn, tp, pp), lambda i: (i, 0, 0)),
            pl.BlockSpec((p + 1, q), lambda i: (0, 0)),
            pl.BlockSpec((q, r), lambda i: (0, 0)),
            pl.BlockSpec((1, r), lambda i: (0, 0)),
        ],
        out_specs=pl.BlockSpec((bn, tp, rp), lambda i: (i, 0, 0)),
        out_shape=jax.ShapeDtypeStruct((n, tp, rp), jnp.float32),
        scratch_shapes=[pltpu.VMEM((pp, rp), jnp.bfloat16),
                        pltpu.VMEM((1, rp), jnp.float32)],
    )(Xp, m1a, M2, b2eff)

    return padded[:, :t, :r]


# speedup vs baseline: 1.5052x; 1.5052x over previous
import jax
import jax.numpy as jnp
from jax.experimental import pallas as pl
from jax.experimental.pallas import tpu as pltpu

SAMPLES_PER_BLOCK = 8


def kernel(x, A, W1, b1, W2, b2):
    n, t, v, ci = x.shape
    k = A.shape[0]
    h = W1.shape[0] // k
    co = W2.shape[0] // k
    p, r = v * ci, v * co
    q = v * h
    tp = (t + 7) // 8 * 8
    pp = 128
    rp = (r + 127) // 128 * 128
    bn = SAMPLES_PER_BLOCK

    W1r = W1.reshape(k, h, ci).transpose(0, 2, 1)
    W2r = W2.reshape(k, co, h).transpose(0, 2, 1)
    M1 = (A[:, :, None, :, None] * W1r[:, None, :, None, :]).sum(0)
    M1 = M1.reshape(p, q)
    M2 = (A[:, :, None, :, None] * W2r[:, None, :, None, :]).sum(0)
    M2 = M2.reshape(q, r)
    S = A.sum(axis=1)
    b1r = b1.reshape(k, h)
    b2r = b2.reshape(k, co)
    B1 = (S.T[:, :, None] * b1r[None, :, :]).sum(1).reshape(1, q)
    b2eff = (S.T[:, :, None] * b2r[None, :, :]).sum(1).reshape(1, r)

    m1a = jnp.concatenate([M1, B1], axis=0)

    X3 = x.reshape(n, t, p)
    Xp = jax.lax.pad(
        X3, jnp.float32(0), ((0, 0, 0), (0, tp - t, 0), (0, pp - p, 0)))

    def gemm_fold_kernel(x_ref, m1a_ref, m2_ref, b2_ref, out_ref,
                         ms_ref, bs_ref):
        i = pl.program_id(0)

        @pl.when(i == 0)
        def _():
            mf = jnp.dot(m1a_ref[...], m2_ref[...],
                         preferred_element_type=jnp.float32)
            ms_ref[...] = jnp.zeros((pp, rp), jnp.bfloat16)
            ms_ref[0:p, 0:r] = mf[0:p].astype(jnp.bfloat16)
            bs_ref[...] = jnp.zeros((1, rp), jnp.float32)
            bs_ref[0:1, 0:r] = mf[p:p + 1] + b2_ref[...]

        mb = ms_ref[...]
        bb = bs_ref[...]
        for j in range(bn):
            xb = x_ref[j].astype(jnp.bfloat16)
            out_ref[j] = (
                jnp.dot(xb, mb, preferred_element_type=jnp.float32) + bb)

    padded = pl.pallas_call(
        gemm_fold_kernel,
        grid=(n // bn,),
        in_specs=[
            pl.BlockSpec((bn, tp, pp), lambda i: (i, 0, 0)),
            pl.BlockSpec((p + 1, q), lambda i: (0, 0)),
            pl.BlockSpec((q, r), lambda i: (0, 0)),
            pl.BlockSpec((1, r), lambda i: (0, 0)),
        ],
        out_specs=pl.BlockSpec((bn, tp, rp), lambda i: (i, 0, 0)),
        out_shape=jax.ShapeDtypeStruct((n, tp, rp), jnp.float32),
        scratch_shapes=[pltpu.VMEM((pp, rp), jnp.bfloat16),
                        pltpu.VMEM((1, rp), jnp.float32)],
    )(Xp, m1a, M2, b2eff)

    return padded[:, :t, :r]
